# baseline (speedup 1.0000x reference)
import functools

import jax
import jax.numpy as jnp
from jax import lax
from jax.experimental import pallas as pl
from jax.experimental.pallas import tpu as pltpu
from jax.experimental.pallas import tpu_sc as plsc

VOCAB = 1000000
EMBED = 64
NUM_CLASSES = 1000
BATCH = 16384

_info = plsc.get_sparse_core_info()
_NC, _NS = _info.num_cores, _info.num_subcores
_NW = _NC * _NS
_BPW = BATCH // _NW
_CHUNK = 128
_NCHUNK = _BPW // _CHUNK

_VB = 32768
_QB = _VB // 4
_NVB = (VOCAB + _VB - 1) // _VB
_PACK_ROWS = _NVB * _QB
_HALF = EMBED // 2


def _pack_body(tt_ref, o_ref):
    x = tt_ref[...]
    ua = lax.bitcast_convert_type(x[:_HALF, :], jnp.uint32)
    ub = lax.bitcast_convert_type(x[_HALF:, :], jnp.uint32)
    p32 = ((ua + 0x8000) >> 16) | ((ub + 0x8000) & jnp.uint32(0xFFFF0000))
    p = lax.bitcast_convert_type(p32, jnp.float32)
    o_ref[...] = jnp.concatenate(
        [p[:, q * _QB:(q + 1) * _QB].T for q in range(4)], axis=1
    )


def _pack_tc(tt):
    return pl.pallas_call(
        _pack_body,
        grid=(_NVB,),
        in_specs=[pl.BlockSpec((EMBED, _VB), lambda i: (0, i))],
        out_specs=pl.BlockSpec((_QB, 4 * _HALF), lambda i: (i, 0)),
        out_shape=jax.ShapeDtypeStruct((_PACK_ROWS, 4 * _HALF), jnp.float32),
    )(tt)


_sc_mesh = plsc.VectorSubcoreMesh(core_axis_name="c", subcore_axis_name="s")


@functools.partial(
    pl.kernel,
    mesh=_sc_mesh,
    out_type=jax.ShapeDtypeStruct((BATCH, 4 * _HALF), jnp.float32),
    scratch_types=[
        pltpu.VMEM((_NCHUNK, _CHUNK), jnp.int32),
        pltpu.VMEM((_BPW, 4 * _HALF), jnp.float32),
        pltpu.SemaphoreType.DMA,
    ],
)
def _gather_sc(idx_hbm, t2_hbm, out_hbm, idx_v, rows_v, sem):
    wid = lax.axis_index("s") * _NC + lax.axis_index("c")
    base = wid * _BPW
    pltpu.sync_copy(idx_hbm.at[wid], idx_v)
    copies = []
    for j in range(_NCHUNK):
        copies.append(
            pltpu.async_copy(
                t2_hbm.at[idx_v.at[j]],
                rows_v.at[pl.ds(j * _CHUNK, _CHUNK)],
                sem,
            )
        )
    for c in copies:
        c.wait()
    pltpu.sync_copy(rows_v, out_hbm.at[pl.ds(base, _BPW)])


_BB = 2048


def _matmul_body(e2_ref, q_ref, w_ref, b_ref, o_ref):
    u = lax.bitcast_convert_type(e2_ref[...], jnp.uint32)
    q = q_ref[...]
    uq = jnp.where(
        q < 2,
        jnp.where(q < 1, u[:, 0:32], u[:, 32:64]),
        jnp.where(q < 3, u[:, 64:96], u[:, 96:128]),
    )
    lo = lax.bitcast_convert_type(uq << 16, jnp.float32)
    hi = lax.bitcast_convert_type(uq & jnp.uint32(0xFFFF0000), jnp.float32)
    x = jnp.concatenate([lo, hi], axis=1)
    o_ref[...] = (
        jnp.dot(x, w_ref[...], preferred_element_type=jnp.float32)
        + b_ref[...]
    )


def _matmul_tc(emb2, quart, W, b2d):
    return pl.pallas_call(
        _matmul_body,
        grid=(BATCH // _BB,),
        in_specs=[
            pl.BlockSpec((_BB, 4 * _HALF), lambda i: (i, 0)),
            pl.BlockSpec((_BB, 1), lambda i: (i, 0)),
            pl.BlockSpec((EMBED, NUM_CLASSES), lambda i: (0, 0)),
            pl.BlockSpec((1, NUM_CLASSES), lambda i: (0, 0)),
        ],
        out_specs=pl.BlockSpec((_BB, NUM_CLASSES), lambda i: (i, 0)),
        out_shape=jax.ShapeDtypeStruct((BATCH, NUM_CLASSES), jnp.float32),
    )(emb2, quart, W, b2d)


def kernel(inputs, table, W, b):
    idx = inputs.astype(jnp.int32)
    t2 = _pack_tc(table.T)
    blk = idx // _VB
    off = idx % _VB
    line = (blk * _QB + off % _QB).reshape(_NW, _NCHUNK, _CHUNK)
    emb2 = _gather_sc(line, t2)
    quart = (off // _QB).reshape(BATCH, 1)
    return _matmul_tc(emb2, quart, W, b.reshape(1, NUM_CLASSES))


# speedup vs baseline: 1.1014x; 1.1014x over previous
import functools

import jax
import jax.numpy as jnp
from jax import lax
from jax.experimental import pallas as pl
from jax.experimental.pallas import tpu as pltpu
from jax.experimental.pallas import tpu_sc as plsc

VOCAB = 1000000
EMBED = 64
NUM_CLASSES = 1000
BATCH = 16384

_info = plsc.get_sparse_core_info()
_NC, _NS = _info.num_cores, _info.num_subcores
_NW = _NC * _NS
_BPW = BATCH // _NW
_CHUNK = 128
_NCHUNK = _BPW // _CHUNK

_VB = 32768
_HB = _VB // 2
_NVB = (VOCAB + _VB - 1) // _VB
_PACK_ROWS = _NVB * _HB


def _pack_body(tt_ref, o_ref):
    x = tt_ref[...]
    o_ref[...] = jnp.concatenate([x[:, :_HB].T, x[:, _HB:].T], axis=1)


def _pack_tc(tt):
    return pl.pallas_call(
        _pack_body,
        grid=(_NVB,),
        in_specs=[pl.BlockSpec((EMBED, _VB), lambda i: (0, i))],
        out_specs=pl.BlockSpec((_HB, 2 * EMBED), lambda i: (i, 0)),
        out_shape=jax.ShapeDtypeStruct((_PACK_ROWS, 2 * EMBED), jnp.float32),
    )(tt)


_sc_mesh = plsc.VectorSubcoreMesh(core_axis_name="c", subcore_axis_name="s")


@functools.partial(
    pl.kernel,
    mesh=_sc_mesh,
    out_type=jax.ShapeDtypeStruct((BATCH, 2 * EMBED), jnp.float32),
    scratch_types=[
        pltpu.VMEM((_NCHUNK, _CHUNK), jnp.int32),
        pltpu.VMEM((_BPW, 2 * EMBED), jnp.float32),
        pltpu.SemaphoreType.DMA,
    ],
)
def _gather_sc(idx_hbm, t2_hbm, out_hbm, idx_v, rows_v, sem):
    wid = lax.axis_index("s") * _NC + lax.axis_index("c")
    base = wid * _BPW
    pltpu.sync_copy(idx_hbm.at[wid], idx_v)
    copies = []
    for j in range(_NCHUNK):
        copies.append(
            pltpu.async_copy(
                t2_hbm.at[idx_v.at[j]],
                rows_v.at[pl.ds(j * _CHUNK, _CHUNK)],
                sem,
            )
        )
    for c in copies:
        c.wait()
    pltpu.sync_copy(rows_v, out_hbm.at[pl.ds(base, _BPW)])


_BB = 4096


def _matmul_body(e2_ref, par_ref, w_ref, b_ref, o_ref):
    e2 = e2_ref[...]
    par = par_ref[...]
    x = jnp.where(par == 1, e2[:, EMBED:], e2[:, :EMBED]).astype(jnp.bfloat16)
    o_ref[...] = (
        jnp.dot(x, w_ref[...], preferred_element_type=jnp.float32)
        + b_ref[...]
    )


def _matmul_tc(emb2, par, Wb, b2d):
    return pl.pallas_call(
        _matmul_body,
        grid=(BATCH // _BB,),
        in_specs=[
            pl.BlockSpec((_BB, 2 * EMBED), lambda i: (i, 0)),
            pl.BlockSpec((_BB, 1), lambda i: (i, 0)),
            pl.BlockSpec((EMBED, NUM_CLASSES), lambda i: (0, 0)),
            pl.BlockSpec((1, NUM_CLASSES), lambda i: (0, 0)),
        ],
        out_specs=pl.BlockSpec((_BB, NUM_CLASSES), lambda i: (i, 0)),
        out_shape=jax.ShapeDtypeStruct((BATCH, NUM_CLASSES), jnp.float32),
    )(emb2, par, Wb, b2d)


def kernel(inputs, table, W, b):
    idx = inputs.astype(jnp.int32)
    t2 = _pack_tc(table.T)
    blk = idx // _VB
    off = idx % _VB
    line = (blk * _HB + off % _HB).reshape(_NW, _NCHUNK, _CHUNK)
    emb2 = _gather_sc(line, t2)
    par = (off >= _HB).astype(jnp.int32).reshape(BATCH, 1)
    return _matmul_tc(
        emb2, par, W.astype(jnp.bfloat16), b.reshape(1, NUM_CLASSES)
    )
